# baseline (speedup 1.0000x reference)
import jax
import jax.numpy as jnp
from jax import lax
from jax.experimental import pallas as pl

_BLK = 512


def _add_kernel(in_ref, table_ref, out_ref):
    out_ref[...] = in_ref[...] + table_ref[...]


def kernel(inputs, pos_table):
    batch, seq_len, out_dim = inputs.shape
    flat = inputs.reshape(batch * seq_len, out_dim)
    tab_blocks = seq_len // _BLK
    grid = (batch * seq_len // _BLK,)
    out = pl.pallas_call(
        _add_kernel,
        grid=grid,
        in_specs=[
            pl.BlockSpec((_BLK, out_dim), lambda i: (i, 0)),
            pl.BlockSpec((_BLK, out_dim),
                         lambda i: (lax.rem(i, tab_blocks), 0)),
        ],
        out_specs=pl.BlockSpec((_BLK, out_dim), lambda i: (i, 0)),
        out_shape=jax.ShapeDtypeStruct(flat.shape, flat.dtype),
    )(flat, pos_table)
    return out.reshape(batch, seq_len, out_dim)


# speedup vs baseline: 1.1920x; 1.1920x over previous
import jax
import jax.numpy as jnp
from jax.experimental import pallas as pl

_BLK = 512


def _add_kernel(in_ref, table_ref, out_ref):
    out_ref[...] = in_ref[...] + table_ref[...]


def kernel(inputs, pos_table):
    batch, seq_len, out_dim = inputs.shape
    flat = inputs.reshape(batch * seq_len, out_dim)
    tab_blocks = seq_len // _BLK
    grid = (tab_blocks, batch)
    out = pl.pallas_call(
        _add_kernel,
        grid=grid,
        in_specs=[
            pl.BlockSpec((_BLK, out_dim),
                         lambda i, j: (j * tab_blocks + i, 0)),
            pl.BlockSpec((_BLK, out_dim), lambda i, j: (i, 0)),
        ],
        out_specs=pl.BlockSpec((_BLK, out_dim),
                               lambda i, j: (j * tab_blocks + i, 0)),
        out_shape=jax.ShapeDtypeStruct(flat.shape, flat.dtype),
    )(flat, pos_table)
    return out.reshape(batch, seq_len, out_dim)


# speedup vs baseline: 1.3269x; 1.1132x over previous
import jax
import jax.numpy as jnp
from jax.experimental import pallas as pl

_SEQ_BLOCK = 256


def _add_kernel(in_ref, table_ref, out_ref):
    out_ref[...] = in_ref[...] + table_ref[...][None, :, :]


def kernel(inputs, pos_table):
    batch, seq_len, out_dim = inputs.shape
    grid = (seq_len // _SEQ_BLOCK,)
    return pl.pallas_call(
        _add_kernel,
        grid=grid,
        in_specs=[
            pl.BlockSpec((batch, _SEQ_BLOCK, out_dim), lambda i: (0, i, 0)),
            pl.BlockSpec((_SEQ_BLOCK, out_dim), lambda i: (i, 0)),
        ],
        out_specs=pl.BlockSpec((batch, _SEQ_BLOCK, out_dim), lambda i: (0, i, 0)),
        out_shape=jax.ShapeDtypeStruct(inputs.shape, inputs.dtype),
    )(inputs, pos_table)
